# baseline (speedup 1.0000x reference)
import functools

import jax
import jax.numpy as jnp
from jax import lax
from jax.experimental import pallas as pl
from jax.experimental.pallas import tpu as pltpu
from jax.experimental.pallas import tpu_sc as plsc

NC = 2
NS = 16
NW = NC * NS
L = 16


def _body(num_rel, n_per_worker, chunk, ent_hbm, rel_hbm, h_hbm, t_hbm,
          r_hbm, out_hbm, hidx, tidx, ridx, nh, nt, nr, hrows, trows, rrows,
          outbuf, sem_h, sem_t, sem_r):
    wid = lax.axis_index("s") * NC + lax.axis_index("c")
    base = wid * n_per_worker
    pltpu.sync_copy(h_hbm.at[pl.ds(base, n_per_worker)], hidx)
    pltpu.sync_copy(t_hbm.at[pl.ds(base, n_per_worker)], tidx)
    pltpu.sync_copy(r_hbm.at[pl.ds(base, n_per_worker)], ridx)

    iota = lax.iota(jnp.int32, L)
    n_chunks = n_per_worker // chunk
    rows_per_chunk = chunk // 64

    def chunk_body(c, _):
        cb = c * chunk
        for j in range(rows_per_chunk):
            off = cb + j * 64
            first = plsc.load_gather(hidx, [jnp.full((L,), off, jnp.int32)])
            m = None
            hv = []
            for k in range(4):
                v = hidx[pl.ds(off + k * L, L)]
                hv.append(v)
                e = v == first
                m = e if m is None else (m & e)
            cnt = plsc.all_reduce_population_count(m)
            is_neg = cnt == L
            for k in range(4):
                sl = pl.ds(off + k * L, L)
                tv = tidx[sl]
                rv = ridx[sl]
                dsl = pl.ds(j * 64 + k * L, L)
                nh[dsl] = jnp.where(is_neg, hv[k], tv)
                nt[dsl] = jnp.where(is_neg, tv, hv[k])
                nr[dsl] = jnp.where(is_neg, rv, rv + num_rel)
        cp_h = pltpu.async_copy(ent_hbm.at[nh], hrows, sem_h)
        cp_t = pltpu.async_copy(ent_hbm.at[nt], trows, sem_t)
        cp_r = pltpu.async_copy(rel_hbm.at[nr], rrows, sem_r)
        cp_h.wait()
        cp_t.wait()
        cp_r.wait()
        for g in range(chunk // L):
            ev = g * L + iota

            def dbody(i, acc):
                for u in range(8):
                    dv = jnp.full((L,), i * 8 + u, jnp.int32)
                    hvv = plsc.load_gather(hrows, [ev, dv])
                    tvv = plsc.load_gather(trows, [ev, dv])
                    rvv = plsc.load_gather(rrows, [ev, dv])
                    acc = acc + hvv * rvv * tvv
                return acc

            acc = lax.fori_loop(0, 8, dbody, jnp.zeros((L,), jnp.float32))
            outbuf[pl.ds(cb + g * L, L)] = acc
        return 0

    lax.fori_loop(0, n_chunks, chunk_body, 0)
    pltpu.sync_copy(outbuf, out_hbm.at[pl.ds(base, n_per_worker)])


def kernel(entity_emb, relation_emb, h_index, t_index, r_index):
    shape = h_index.shape
    total = h_index.size
    num_rel = relation_emb.shape[0] // 2
    dim = entity_emb.shape[1]
    n_per_worker = total // NW
    chunk = 128

    mesh = plsc.VectorSubcoreMesh(core_axis_name="c", subcore_axis_name="s")
    body = functools.partial(_body, num_rel, n_per_worker, chunk)
    run = pl.kernel(
        body,
        out_type=jax.ShapeDtypeStruct((total,), jnp.float32),
        mesh=mesh,
        compiler_params=pltpu.CompilerParams(
            needs_layout_passes=False, use_tc_tiling_on_sc=False),
        scratch_types=[
            pltpu.VMEM((n_per_worker,), jnp.int32),
            pltpu.VMEM((n_per_worker,), jnp.int32),
            pltpu.VMEM((n_per_worker,), jnp.int32),
            pltpu.VMEM((chunk,), jnp.int32),
            pltpu.VMEM((chunk,), jnp.int32),
            pltpu.VMEM((chunk,), jnp.int32),
            pltpu.VMEM((chunk, dim), jnp.float32),
            pltpu.VMEM((chunk, dim), jnp.float32),
            pltpu.VMEM((chunk, dim), jnp.float32),
            pltpu.VMEM((n_per_worker,), jnp.float32),
            pltpu.SemaphoreType.DMA,
            pltpu.SemaphoreType.DMA,
            pltpu.SemaphoreType.DMA,
        ],
    )
    out = run(entity_emb, relation_emb, h_index.reshape(-1),
              t_index.reshape(-1), r_index.reshape(-1))
    return out.reshape(shape)


# speedup vs baseline: 1.0819x; 1.0819x over previous
import functools

import jax
import jax.numpy as jnp
from jax import lax
from jax.experimental import pallas as pl
from jax.experimental.pallas import tpu as pltpu
from jax.experimental.pallas import tpu_sc as plsc

NC = 2
NS = 16
NW = NC * NS
L = 16
CHUNK = 128


def _body(num_rel, n_per_worker, ent_hbm, rel_hbm, h_hbm, t_hbm,
          r_hbm, out_hbm, hidx, tidx, ridx, rel, hb0, tb0, hb1, tb1,
          outbuf, sh0, st0, sh1, st1):
    wid = lax.axis_index("s") * NC + lax.axis_index("c")
    base = wid * n_per_worker
    pltpu.sync_copy(h_hbm.at[pl.ds(base, n_per_worker)], hidx)
    pltpu.sync_copy(t_hbm.at[pl.ds(base, n_per_worker)], tidx)
    pltpu.sync_copy(r_hbm.at[pl.ds(base, n_per_worker)], ridx)
    pltpu.sync_copy(rel_hbm, rel)

    iota = lax.iota(jnp.int32, L)
    n_chunks = n_per_worker // CHUNK

    def row_body(b, _):
        off = b * 64
        first = plsc.load_gather(hidx, [jnp.full((L,), off, jnp.int32)])
        hv, tv, rv, m = [], [], [], None
        for k in range(4):
            sl = pl.ds(off + k * L, L)
            hv.append(hidx[sl])
            tv.append(tidx[sl])
            rv.append(ridx[sl])
            e = hv[k] == first
            m = e if m is None else (m & e)
        is_neg = plsc.all_reduce_population_count(m) == L
        for k in range(4):
            sl = pl.ds(off + k * L, L)
            hidx[sl] = jnp.where(is_neg, hv[k], tv[k])
            tidx[sl] = jnp.where(is_neg, tv[k], hv[k])
            ridx[sl] = jnp.where(is_neg, rv[k], rv[k] + num_rel)
        return 0

    lax.fori_loop(0, n_per_worker // 64, row_body, 0)

    def fire(cb, hb, tb, sh, st):
        pltpu.async_copy(ent_hbm.at[hidx.at[pl.ds(cb, CHUNK)]], hb, sh)
        pltpu.async_copy(ent_hbm.at[tidx.at[pl.ds(cb, CHUNK)]], tb, st)

    def drain(hb, tb, sh, st):
        pltpu.make_async_copy(
            ent_hbm.at[hidx.at[pl.ds(0, CHUNK)]], hb, sh).wait()
        pltpu.make_async_copy(
            ent_hbm.at[tidx.at[pl.ds(0, CHUNK)]], tb, st).wait()

    def compute(cb, hb, tb):
        for g in range(CHUNK // L):
            ev = iota + g * L
            nrv = ridx[pl.ds(cb + g * L, L)]

            def dbody(i, acc):
                for u in range(16):
                    dv = jnp.full((L,), i * 16 + u, jnp.int32)
                    hvv = plsc.load_gather(hb, [ev, dv])
                    tvv = plsc.load_gather(tb, [ev, dv])
                    rvv = plsc.load_gather(rel, [nrv, dv])
                    acc = acc + hvv * tvv * rvv
                return acc

            acc = lax.fori_loop(0, 4, dbody, jnp.zeros((L,), jnp.float32))
            outbuf[pl.ds(cb + g * L, L)] = acc

    fire(0, hb0, tb0, sh0, st0)

    def loop_body(c2, _):
        cb = c2 * (2 * CHUNK)
        fire(cb + CHUNK, hb1, tb1, sh1, st1)
        drain(hb0, tb0, sh0, st0)
        compute(cb, hb0, tb0)

        @pl.when(c2 < n_chunks // 2 - 1)
        def _():
            fire(cb + 2 * CHUNK, hb0, tb0, sh0, st0)

        drain(hb1, tb1, sh1, st1)
        compute(cb + CHUNK, hb1, tb1)
        return 0

    lax.fori_loop(0, n_chunks // 2, loop_body, 0)
    pltpu.sync_copy(outbuf, out_hbm.at[pl.ds(base, n_per_worker)])


def kernel(entity_emb, relation_emb, h_index, t_index, r_index):
    shape = h_index.shape
    total = h_index.size
    num_rel = relation_emb.shape[0] // 2
    dim = entity_emb.shape[1]
    n_per_worker = total // NW

    mesh = plsc.VectorSubcoreMesh(core_axis_name="c", subcore_axis_name="s")
    body = functools.partial(_body, num_rel, n_per_worker)
    run = pl.kernel(
        body,
        out_type=jax.ShapeDtypeStruct((total,), jnp.float32),
        mesh=mesh,
        compiler_params=pltpu.CompilerParams(
            needs_layout_passes=False, use_tc_tiling_on_sc=False),
        scratch_types=[
            pltpu.VMEM((n_per_worker,), jnp.int32),
            pltpu.VMEM((n_per_worker,), jnp.int32),
            pltpu.VMEM((n_per_worker,), jnp.int32),
            pltpu.VMEM((2 * num_rel, dim), jnp.float32),
            pltpu.VMEM((CHUNK, dim), jnp.float32),
            pltpu.VMEM((CHUNK, dim), jnp.float32),
            pltpu.VMEM((CHUNK, dim), jnp.float32),
            pltpu.VMEM((CHUNK, dim), jnp.float32),
            pltpu.VMEM((n_per_worker,), jnp.float32),
            pltpu.SemaphoreType.DMA,
            pltpu.SemaphoreType.DMA,
            pltpu.SemaphoreType.DMA,
            pltpu.SemaphoreType.DMA,
        ],
    )
    out = run(entity_emb, relation_emb, h_index.reshape(-1),
              t_index.reshape(-1), r_index.reshape(-1))
    return out.reshape(shape)


# speedup vs baseline: 1.0931x; 1.0103x over previous
import functools

import jax
import jax.numpy as jnp
from jax import lax
from jax.experimental import pallas as pl
from jax.experimental.pallas import tpu as pltpu
from jax.experimental.pallas import tpu_sc as plsc

NC = 2
NS = 16
NW = NC * NS
L = 16
CHUNK = 64
NBUF = 4


def _body(num_rel, n_per_worker, ent_hbm, rel_hbm, h_hbm, t_hbm,
          r_hbm, out_hbm, hidx, tidx, ridx, rel, cidx, rows, score,
          gsems, ssems):
    wid = lax.axis_index("s") * NC + lax.axis_index("c")
    base = wid * n_per_worker
    pltpu.sync_copy(h_hbm.at[pl.ds(base, n_per_worker)], hidx)
    pltpu.sync_copy(t_hbm.at[pl.ds(base, n_per_worker)], tidx)
    pltpu.sync_copy(r_hbm.at[pl.ds(base, n_per_worker)], ridx)
    pltpu.sync_copy(rel_hbm, rel)

    iota = lax.iota(jnp.int32, L)
    n_chunks = n_per_worker // CHUNK

    def prep_fire(k, b):
        off = k * CHUNK
        first = plsc.load_gather(hidx, [jnp.full((L,), off, jnp.int32)])
        hv, tv, rv, m = [], [], [], None
        for j in range(4):
            sl = pl.ds(off + j * L, L)
            hv.append(hidx[sl])
            tv.append(tidx[sl])
            rv.append(ridx[sl])
            e = hv[j] == first
            m = e if m is None else (m & e)
        is_neg = plsc.all_reduce_population_count(m) == L
        for j in range(4):
            cidx[b][pl.ds(j * L, L)] = jnp.where(is_neg, hv[j], tv[j])
            cidx[b][pl.ds(CHUNK + j * L, L)] = jnp.where(is_neg, tv[j], hv[j])
            ridx[pl.ds(off + j * L, L)] = jnp.where(is_neg, rv[j],
                                                    rv[j] + num_rel)
        pltpu.async_copy(ent_hbm.at[cidx[b]], rows[b], gsems[b])

    def wait_gather(b):
        pltpu.make_async_copy(ent_hbm.at[cidx[b]], rows[b], gsems[b]).wait()

    def wait_score(b):
        pltpu.make_async_copy(score[b], out_hbm.at[pl.ds(0, CHUNK)],
                              ssems[b]).wait()

    def compute(k, b):
        off = k * CHUNK
        for g in range(CHUNK // L):
            ev = iota + g * L
            evt = ev + CHUNK
            nrv = ridx[pl.ds(off + g * L, L)]

            def dbody(i, acc):
                for u in range(16):
                    dv = jnp.full((L,), i * 16 + u, jnp.int32)
                    hvv = plsc.load_gather(rows[b], [ev, dv])
                    tvv = plsc.load_gather(rows[b], [evt, dv])
                    rvv = plsc.load_gather(rel, [nrv, dv])
                    acc = acc + hvv * tvv * rvv
                return acc

            acc = lax.fori_loop(0, 4, dbody, jnp.zeros((L,), jnp.float32))
            score[b][pl.ds(g * L, L)] = acc
        pltpu.async_copy(score[b], out_hbm.at[pl.ds(base + off, CHUNK)],
                         ssems[b])

    for b in range(NBUF - 1):
        prep_fire(b, b)

    def outer(c, _):
        k0 = c * NBUF
        for b in range(NBUF):
            k = k0 + b
            nb = (b + NBUF - 1) % NBUF

            @pl.when(k + NBUF - 1 < n_chunks)
            def _():
                prep_fire(k + NBUF - 1, nb)

            wait_gather(b)

            @pl.when(c > 0)
            def _():
                wait_score(b)

            compute(k, b)
        return 0

    lax.fori_loop(0, n_chunks // NBUF, outer, 0)
    for b in range(NBUF):
        wait_score(b)


def kernel(entity_emb, relation_emb, h_index, t_index, r_index):
    shape = h_index.shape
    total = h_index.size
    num_rel = relation_emb.shape[0] // 2
    dim = entity_emb.shape[1]
    n_per_worker = total // NW

    mesh = plsc.VectorSubcoreMesh(core_axis_name="c", subcore_axis_name="s")
    body = functools.partial(_body, num_rel, n_per_worker)
    run = pl.kernel(
        body,
        out_type=jax.ShapeDtypeStruct((total,), jnp.float32),
        mesh=mesh,
        compiler_params=pltpu.CompilerParams(
            needs_layout_passes=False, use_tc_tiling_on_sc=False),
        scratch_types=[
            pltpu.VMEM((n_per_worker,), jnp.int32),
            pltpu.VMEM((n_per_worker,), jnp.int32),
            pltpu.VMEM((n_per_worker,), jnp.int32),
            pltpu.VMEM((2 * num_rel, dim), jnp.float32),
            [pltpu.VMEM((2 * CHUNK,), jnp.int32)] * NBUF,
            [pltpu.VMEM((2 * CHUNK, dim), jnp.float32)] * NBUF,
            [pltpu.VMEM((CHUNK,), jnp.float32)] * NBUF,
            [pltpu.SemaphoreType.DMA] * NBUF,
            [pltpu.SemaphoreType.DMA] * NBUF,
        ],
    )
    out = run(entity_emb, relation_emb, h_index.reshape(-1),
              t_index.reshape(-1), r_index.reshape(-1))
    return out.reshape(shape)


# speedup vs baseline: 1.1055x; 1.0114x over previous
import functools

import jax
import jax.numpy as jnp
from jax import lax
from jax.experimental import pallas as pl
from jax.experimental.pallas import tpu as pltpu
from jax.experimental.pallas import tpu_sc as plsc

NC = 2
NS = 16
NW = NC * NS
L = 16
CHUNK = 64
NBUF = 4


def _body(num_rel, n_per_worker, ent_hbm, rel_hbm, h_hbm, t_hbm,
          r_hbm, out_hbm, hidx, tidx, ridx, rel, cidx, rows, score,
          gsems, ssems):
    wid = lax.axis_index("s") * NC + lax.axis_index("c")
    base = wid * n_per_worker
    pltpu.sync_copy(h_hbm.at[pl.ds(base, n_per_worker)], hidx)
    pltpu.sync_copy(t_hbm.at[pl.ds(base, n_per_worker)], tidx)
    pltpu.sync_copy(r_hbm.at[pl.ds(base, n_per_worker)], ridx)
    pltpu.sync_copy(rel_hbm, rel)

    iota = lax.iota(jnp.int32, L)
    n_chunks = n_per_worker // CHUNK

    def prep_fire(k, b):
        off = k * CHUNK
        first = plsc.load_gather(hidx, [jnp.full((L,), off, jnp.int32)])
        hv, tv, rv, m = [], [], [], None
        for j in range(4):
            sl = pl.ds(off + j * L, L)
            hv.append(hidx[sl])
            tv.append(tidx[sl])
            rv.append(ridx[sl])
            e = hv[j] == first
            m = e if m is None else (m & e)
        is_neg = plsc.all_reduce_population_count(m) == L
        for j in range(4):
            cidx[b][pl.ds(j * L, L)] = jnp.where(is_neg, hv[j], tv[j])
            cidx[b][pl.ds(CHUNK + j * L, L)] = jnp.where(is_neg, tv[j], hv[j])
            ridx[pl.ds(off + j * L, L)] = jnp.where(is_neg, rv[j],
                                                    rv[j] + num_rel)
        pltpu.async_copy(ent_hbm.at[cidx[b]], rows[b], gsems[b])

    def wait_gather(b):
        pltpu.make_async_copy(ent_hbm.at[cidx[b]], rows[b], gsems[b]).wait()

    def wait_score(b):
        pltpu.make_async_copy(score[b], out_hbm.at[pl.ds(0, CHUNK)],
                              ssems[b]).wait()

    def compute(k, b):
        off = k * CHUNK
        for g in range(CHUNK // L):
            ev = iota + g * L
            evt = ev + CHUNK
            nrv = ridx[pl.ds(off + g * L, L)]

            def dbody(i, accs):
                accs = list(accs)
                for u in range(16):
                    dv = jnp.full((L,), i * 16 + u, jnp.int32)
                    hvv = plsc.load_gather(rows[b], [ev, dv])
                    tvv = plsc.load_gather(rows[b], [evt, dv])
                    rvv = plsc.load_gather(rel, [nrv, dv])
                    accs[u % 4] = accs[u % 4] + hvv * tvv * rvv
                return tuple(accs)

            z = jnp.zeros((L,), jnp.float32)
            accs = lax.fori_loop(0, 4, dbody, (z, z, z, z))
            score[b][pl.ds(g * L, L)] = ((accs[0] + accs[1])
                                         + (accs[2] + accs[3]))
        pltpu.async_copy(score[b], out_hbm.at[pl.ds(base + off, CHUNK)],
                         ssems[b])

    for b in range(NBUF - 1):
        prep_fire(b, b)

    def outer(c, _):
        k0 = c * NBUF
        for b in range(NBUF):
            k = k0 + b
            nb = (b + NBUF - 1) % NBUF

            @pl.when(k + NBUF - 1 < n_chunks)
            def _():
                prep_fire(k + NBUF - 1, nb)

            wait_gather(b)

            @pl.when(c > 0)
            def _():
                wait_score(b)

            compute(k, b)
        return 0

    lax.fori_loop(0, n_chunks // NBUF, outer, 0)
    for b in range(NBUF):
        wait_score(b)


def kernel(entity_emb, relation_emb, h_index, t_index, r_index):
    shape = h_index.shape
    total = h_index.size
    num_rel = relation_emb.shape[0] // 2
    dim = entity_emb.shape[1]
    n_per_worker = total // NW

    mesh = plsc.VectorSubcoreMesh(core_axis_name="c", subcore_axis_name="s")
    body = functools.partial(_body, num_rel, n_per_worker)
    run = pl.kernel(
        body,
        out_type=jax.ShapeDtypeStruct((total,), jnp.float32),
        mesh=mesh,
        compiler_params=pltpu.CompilerParams(
            needs_layout_passes=False, use_tc_tiling_on_sc=False,
            disable_bounds_checks=True, disable_semaphore_checks=True),
        scratch_types=[
            pltpu.VMEM((n_per_worker,), jnp.int32),
            pltpu.VMEM((n_per_worker,), jnp.int32),
            pltpu.VMEM((n_per_worker,), jnp.int32),
            pltpu.VMEM((2 * num_rel, dim), jnp.float32),
            [pltpu.VMEM((2 * CHUNK,), jnp.int32)] * NBUF,
            [pltpu.VMEM((2 * CHUNK, dim), jnp.float32)] * NBUF,
            [pltpu.VMEM((CHUNK,), jnp.float32)] * NBUF,
            [pltpu.SemaphoreType.DMA] * NBUF,
            [pltpu.SemaphoreType.DMA] * NBUF,
        ],
    )
    out = run(entity_emb, relation_emb, h_index.reshape(-1),
              t_index.reshape(-1), r_index.reshape(-1))
    return out.reshape(shape)


# speedup vs baseline: 1.8740x; 1.6951x over previous
import functools

import jax
import jax.numpy as jnp
from jax import lax
from jax.experimental import pallas as pl
from jax.experimental.pallas import tpu as pltpu
from jax.experimental.pallas import tpu_sc as plsc

NC = 2
NS = 16
NW = NC * NS
L = 16
CHUNK = 64
NBUF = 4


def _body(num_rel, n_per_worker, ent_hbm, rel_hbm, h_hbm, t_hbm,
          r_hbm, out_hbm, hidx, tidx, ridx, rel, cidx, rows, score,
          gsems, ssems):
    wid = lax.axis_index("s") * NC + lax.axis_index("c")
    base = wid * n_per_worker
    pltpu.sync_copy(h_hbm.at[pl.ds(base, n_per_worker)], hidx)
    pltpu.sync_copy(t_hbm.at[pl.ds(base, n_per_worker)], tidx)
    pltpu.sync_copy(r_hbm.at[pl.ds(base, n_per_worker)], ridx)
    pltpu.sync_copy(rel_hbm, rel)

    iota = lax.iota(jnp.int32, L)
    n_chunks = n_per_worker // CHUNK

    def prep_fire(k, b):
        off = k * CHUNK
        first = plsc.load_gather(hidx, [jnp.full((L,), off, jnp.int32)])
        hv, tv, rv, m = [], [], [], None
        for j in range(4):
            sl = pl.ds(off + j * L, L)
            hv.append(hidx[sl])
            tv.append(tidx[sl])
            rv.append(ridx[sl])
            e = hv[j] == first
            m = e if m is None else (m & e)
        is_neg = plsc.all_reduce_population_count(m) == L
        for j in range(4):
            cidx[b][pl.ds(j * L, L)] = jnp.where(is_neg, hv[j], tv[j])
            cidx[b][pl.ds(CHUNK + j * L, L)] = jnp.where(is_neg, tv[j], hv[j])
            ridx[pl.ds(off + j * L, L)] = jnp.where(is_neg, rv[j],
                                                    rv[j] + num_rel)
        pltpu.async_copy(ent_hbm.at[cidx[b]], rows[b], gsems[b])

    def wait_gather(b):
        pltpu.make_async_copy(ent_hbm.at[cidx[b]], rows[b], gsems[b]).wait()

    def wait_score(b):
        pltpu.make_async_copy(score[b], out_hbm.at[pl.ds(0, CHUNK)],
                              ssems[b]).wait()

    def compute(k, b):
        off = k * CHUNK
        zi = jnp.zeros((L,), jnp.int32)
        dim = 64
        for g in range(CHUNK // L):
            ev = iota + g * L
            nrv = ridx[pl.ds(off + g * L, L)]
            hbase = ev * dim
            tbase = hbase + CHUNK * dim
            rbase = nrv * dim

            def dbody(i, carry):
                offv, a0, a1, a2, a3 = carry
                accs = [a0, a1, a2, a3]
                for u in range(16):
                    hvv = plsc.load_gather(rows[b], [zi, hbase + offv])
                    tvv = plsc.load_gather(rows[b], [zi, tbase + offv])
                    rvv = plsc.load_gather(rel, [zi, rbase + offv])
                    accs[u % 4] = accs[u % 4] + hvv * tvv * rvv
                    offv = (offv + 1) & (dim - 1)
                return (offv, *accs)

            z = jnp.zeros((L,), jnp.float32)
            res = lax.fori_loop(0, 4, dbody, (iota, z, z, z, z))
            score[b][pl.ds(g * L, L)] = ((res[1] + res[2])
                                         + (res[3] + res[4]))
        pltpu.async_copy(score[b], out_hbm.at[pl.ds(base + off, CHUNK)],
                         ssems[b])

    for b in range(NBUF - 1):
        prep_fire(b, b)

    def outer(c, _):
        k0 = c * NBUF
        for b in range(NBUF):
            k = k0 + b
            nb = (b + NBUF - 1) % NBUF

            @pl.when(k + NBUF - 1 < n_chunks)
            def _():
                prep_fire(k + NBUF - 1, nb)

            wait_gather(b)

            @pl.when(c > 0)
            def _():
                wait_score(b)

            compute(k, b)
        return 0

    lax.fori_loop(0, n_chunks // NBUF, outer, 0)
    for b in range(NBUF):
        wait_score(b)


def kernel(entity_emb, relation_emb, h_index, t_index, r_index):
    shape = h_index.shape
    total = h_index.size
    num_rel = relation_emb.shape[0] // 2
    dim = entity_emb.shape[1]
    n_per_worker = total // NW

    mesh = plsc.VectorSubcoreMesh(core_axis_name="c", subcore_axis_name="s")
    body = functools.partial(_body, num_rel, n_per_worker)
    run = pl.kernel(
        body,
        out_type=jax.ShapeDtypeStruct((total,), jnp.float32),
        mesh=mesh,
        compiler_params=pltpu.CompilerParams(
            needs_layout_passes=False, use_tc_tiling_on_sc=False,
            disable_bounds_checks=True, disable_semaphore_checks=True),
        scratch_types=[
            pltpu.VMEM((n_per_worker,), jnp.int32),
            pltpu.VMEM((n_per_worker,), jnp.int32),
            pltpu.VMEM((n_per_worker,), jnp.int32),
            pltpu.VMEM((2 * num_rel, dim), jnp.float32),
            [pltpu.VMEM((2 * CHUNK,), jnp.int32)] * NBUF,
            [pltpu.VMEM((2 * CHUNK, dim), jnp.float32)] * NBUF,
            [pltpu.VMEM((CHUNK,), jnp.float32)] * NBUF,
            [pltpu.SemaphoreType.DMA] * NBUF,
            [pltpu.SemaphoreType.DMA] * NBUF,
        ],
    )
    out = run(entity_emb, relation_emb, h_index.reshape(-1),
              t_index.reshape(-1), r_index.reshape(-1))
    return out.reshape(shape)


# speedup vs baseline: 2.0400x; 1.0886x over previous
import functools

import jax
import jax.numpy as jnp
from jax import lax
from jax.experimental import pallas as pl
from jax.experimental.pallas import tpu as pltpu
from jax.experimental.pallas import tpu_sc as plsc

NC = 2
NS = 16
NW = NC * NS
L = 16
CHUNK = 64
NBUF = 2


def _body(num_rel, n_per_worker, ent_hbm, rel_hbm, h_hbm, t_hbm,
          r_hbm, out_hbm, hidx, tidx, ridx, rel, cidx, rows, score,
          gsems, ssems):
    wid = lax.axis_index("s") * NC + lax.axis_index("c")
    base = wid * n_per_worker
    pltpu.sync_copy(h_hbm.at[pl.ds(base, n_per_worker)], hidx)
    pltpu.sync_copy(t_hbm.at[pl.ds(base, n_per_worker)], tidx)
    pltpu.sync_copy(r_hbm.at[pl.ds(base, n_per_worker)], ridx)
    pltpu.sync_copy(rel_hbm, rel)

    iota = lax.iota(jnp.int32, L)
    n_chunks = n_per_worker // CHUNK

    def prep_fire(k, b):
        off = k * CHUNK
        first = plsc.load_gather(hidx, [jnp.full((L,), off, jnp.int32)])
        hv, tv, rv, m = [], [], [], None
        for j in range(4):
            sl = pl.ds(off + j * L, L)
            hv.append(hidx[sl])
            tv.append(tidx[sl])
            rv.append(ridx[sl])
            e = hv[j] == first
            m = e if m is None else (m & e)
        is_neg = plsc.all_reduce_population_count(m) == L
        for j in range(4):
            cidx[b][pl.ds(j * L, L)] = jnp.where(is_neg, hv[j], tv[j])
            cidx[b][pl.ds(CHUNK + j * L, L)] = jnp.where(is_neg, tv[j], hv[j])
            ridx[pl.ds(off + j * L, L)] = jnp.where(is_neg, rv[j],
                                                    rv[j] + num_rel)
        pltpu.async_copy(ent_hbm.at[cidx[b]], rows[b], gsems[b])

    def wait_gather(b):
        pltpu.make_async_copy(ent_hbm.at[cidx[b]], rows[b],
                              gsems[b]).wait()

    def wait_score(b):
        pltpu.make_async_copy(score[b], out_hbm.at[pl.ds(0, CHUNK)],
                              ssems[b]).wait()

    def compute(k, b):
        off = k * CHUNK
        zi = jnp.zeros((L,), jnp.int32)
        dim = 64
        for g in range(CHUNK // L):
            ev = iota + g * L
            nrv = ridx[pl.ds(off + g * L, L)]
            hbase = ev * 128
            tbase = hbase + CHUNK * 128
            rbase = nrv * dim

            def dbody(i, carry):
                offv, a0, a1, a2, a3 = carry
                accs = [a0, a1, a2, a3]
                for u in range(16):
                    hvv = plsc.load_gather(rows[b], [zi, hbase + offv])
                    tvv = plsc.load_gather(rows[b], [zi, tbase + offv])
                    rvv = plsc.load_gather(rel, [zi, rbase + offv])
                    accs[u % 4] = accs[u % 4] + hvv * tvv * rvv
                    offv = (offv + 1) & (dim - 1)
                return (offv, *accs)

            z = jnp.zeros((L,), jnp.float32)
            res = lax.fori_loop(0, 4, dbody, (iota, z, z, z, z))
            score[b][pl.ds(g * L, L)] = ((res[1] + res[2])
                                         + (res[3] + res[4]))
        pltpu.async_copy(score[b], out_hbm.at[pl.ds(base + off, CHUNK)],
                         ssems[b])

    for b in range(NBUF - 1):
        prep_fire(b, b)

    def outer(c, _):
        k0 = c * NBUF
        for b in range(NBUF):
            k = k0 + b
            nb = (b + NBUF - 1) % NBUF

            @pl.when(k + NBUF - 1 < n_chunks)
            def _():
                prep_fire(k + NBUF - 1, nb)

            wait_gather(b)

            @pl.when(c > 0)
            def _():
                wait_score(b)

            compute(k, b)
        return 0

    lax.fori_loop(0, n_chunks // NBUF, outer, 0)
    for b in range(NBUF):
        wait_score(b)


def kernel(entity_emb, relation_emb, h_index, t_index, r_index):
    shape = h_index.shape
    total = h_index.size
    num_rel = relation_emb.shape[0] // 2
    dim = entity_emb.shape[1]
    n_per_worker = total // NW

    mesh = plsc.VectorSubcoreMesh(core_axis_name="c", subcore_axis_name="s")
    body = functools.partial(_body, num_rel, n_per_worker)
    run = pl.kernel(
        body,
        out_type=jax.ShapeDtypeStruct((total,), jnp.float32),
        mesh=mesh,
        compiler_params=pltpu.CompilerParams(
            needs_layout_passes=False, use_tc_tiling_on_sc=False,
            disable_bounds_checks=True, disable_semaphore_checks=True),
        scratch_types=[
            pltpu.VMEM((n_per_worker,), jnp.int32),
            pltpu.VMEM((n_per_worker,), jnp.int32),
            pltpu.VMEM((n_per_worker,), jnp.int32),
            pltpu.VMEM((2 * num_rel, dim), jnp.float32),
            [pltpu.VMEM((2 * CHUNK,), jnp.int32)] * NBUF,
            [pltpu.VMEM((2 * CHUNK, 128), jnp.float32)] * NBUF,
            [pltpu.VMEM((CHUNK,), jnp.float32)] * NBUF,
            [pltpu.SemaphoreType.DMA] * NBUF,
            [pltpu.SemaphoreType.DMA] * NBUF,
        ],
    )
    ent128 = jnp.pad(entity_emb, ((0, 0), (0, 64)))
    out = run(ent128, relation_emb, h_index.reshape(-1),
              t_index.reshape(-1), r_index.reshape(-1))
    return out.reshape(shape)


# speedup vs baseline: 2.1613x; 1.0595x over previous
import functools

import jax
import jax.numpy as jnp
from jax import lax
from jax.experimental import pallas as pl
from jax.experimental.pallas import tpu as pltpu
from jax.experimental.pallas import tpu_sc as plsc

NC = 2
NS = 16
NW = NC * NS
L = 16
CHUNK = 64
NBUF = 2


def _body(num_rel, n_per_worker, ent_hbm, rel_hbm, h_hbm, t_hbm,
          r_hbm, out_hbm, hidx, tidx, ridx, rel, cidx, rows, score,
          gsems, ssems):
    wid = lax.axis_index("s") * NC + lax.axis_index("c")
    base = wid * n_per_worker
    pltpu.sync_copy(h_hbm.at[pl.ds(base, n_per_worker)], hidx)
    pltpu.sync_copy(t_hbm.at[pl.ds(base, n_per_worker)], tidx)
    pltpu.sync_copy(r_hbm.at[pl.ds(base, n_per_worker)], ridx)
    pltpu.sync_copy(rel_hbm, rel)

    iota = lax.iota(jnp.int32, L)
    n_chunks = n_per_worker // CHUNK

    def prep_fire(k, b):
        off = k * CHUNK
        first = plsc.load_gather(hidx, [jnp.full((L,), off, jnp.int32)])
        hv, tv, rv, m = [], [], [], None
        for j in range(4):
            sl = pl.ds(off + j * L, L)
            hv.append(hidx[sl])
            tv.append(tidx[sl])
            rv.append(ridx[sl])
            e = hv[j] == first
            m = e if m is None else (m & e)
        is_neg = plsc.all_reduce_population_count(m) == L
        for j in range(4):
            nh = jnp.where(is_neg, hv[j], tv[j])
            nt = jnp.where(is_neg, tv[j], hv[j])
            cidx[b][pl.ds(j * L, L)] = nh + nh
            cidx[b][pl.ds(CHUNK + j * L, L)] = nt + nt
            ridx[pl.ds(off + j * L, L)] = jnp.where(is_neg, rv[j],
                                                    rv[j] + num_rel)
        pltpu.async_copy(ent_hbm.at[cidx[b]], rows[b], gsems[b])

    def wait_gather(b):
        pltpu.make_async_copy(ent_hbm.at[cidx[b]], rows[b], gsems[b]).wait()

    def wait_score(b):
        pltpu.make_async_copy(score[b], out_hbm.at[pl.ds(0, CHUNK)],
                              ssems[b]).wait()

    def compute(k, b):
        off = k * CHUNK
        zi = jnp.zeros((L,), jnp.int32)
        dim = 64
        for g in range(CHUNK // L):
            ev = iota + g * L
            nrv = ridx[pl.ds(off + g * L, L)]
            hbase = ev * 64
            tbase = hbase + CHUNK * 64
            rbase = nrv * dim

            def dbody(i, carry):
                offv, a0, a1, a2, a3 = carry
                accs = [a0, a1, a2, a3]
                for u in range(16):
                    hvv = plsc.load_gather(rows[b], [zi, hbase + offv])
                    tvv = plsc.load_gather(rows[b], [zi, tbase + offv])
                    rvv = plsc.load_gather(rel, [zi, rbase + offv])
                    accs[u % 4] = accs[u % 4] + hvv * tvv * rvv
                    offv = (offv + 1) & (dim - 1)
                return (offv, *accs)

            z = jnp.zeros((L,), jnp.float32)
            res = lax.fori_loop(0, 4, dbody, (iota, z, z, z, z))
            score[b][pl.ds(g * L, L)] = ((res[1] + res[2])
                                         + (res[3] + res[4]))
        pltpu.async_copy(score[b], out_hbm.at[pl.ds(base + off, CHUNK)],
                         ssems[b])

    for b in range(NBUF - 1):
        prep_fire(b, b)

    def outer(c, _):
        k0 = c * NBUF
        for b in range(NBUF):
            k = k0 + b
            nb = (b + NBUF - 1) % NBUF

            @pl.when(k + NBUF - 1 < n_chunks)
            def _():
                prep_fire(k + NBUF - 1, nb)

            wait_gather(b)

            @pl.when(c > 0)
            def _():
                wait_score(b)

            compute(k, b)
        return 0

    lax.fori_loop(0, n_chunks // NBUF, outer, 0)
    for b in range(NBUF):
        wait_score(b)


def kernel(entity_emb, relation_emb, h_index, t_index, r_index):
    shape = h_index.shape
    total = h_index.size
    num_rel = relation_emb.shape[0] // 2
    dim = entity_emb.shape[1]
    n_per_worker = total // NW

    mesh = plsc.VectorSubcoreMesh(core_axis_name="c", subcore_axis_name="s")
    body = functools.partial(_body, num_rel, n_per_worker)
    run = pl.kernel(
        body,
        out_type=jax.ShapeDtypeStruct((total,), jnp.float32),
        mesh=mesh,
        compiler_params=pltpu.CompilerParams(
            needs_layout_passes=False, use_tc_tiling_on_sc=False,
            disable_bounds_checks=True, disable_semaphore_checks=True),
        scratch_types=[
            pltpu.VMEM((n_per_worker,), jnp.int32),
            pltpu.VMEM((n_per_worker,), jnp.int32),
            pltpu.VMEM((n_per_worker,), jnp.int32),
            pltpu.VMEM((2 * num_rel, dim), jnp.float32),
            [pltpu.VMEM((2 * CHUNK,), jnp.int32)] * NBUF,
            [pltpu.VMEM((2 * CHUNK, 64), jnp.float32)] * NBUF,
            [pltpu.VMEM((CHUNK,), jnp.float32)] * NBUF,
            [pltpu.SemaphoreType.DMA] * NBUF,
            [pltpu.SemaphoreType.DMA] * NBUF,
        ],
    )
    ent2 = jnp.pad(entity_emb, ((0, 0), (0, 64))).reshape(
        2 * entity_emb.shape[0], 64)
    out = run(ent2, relation_emb, h_index.reshape(-1),
              t_index.reshape(-1), r_index.reshape(-1))
    return out.reshape(shape)
